# baseline (speedup 1.0000x reference)
import jax
import jax.numpy as jnp
from jax.experimental import pallas as pl

_TOP_K1 = 1024
_TOP_K2 = 512
_NUM_HEADS = 16


def _topk_select(vals_ref, vals_from, top_k):
    k = min(top_k, vals_ref.shape[1])
    refs, indices = jax.lax.top_k(vals_ref, k)
    vals = jnp.take_along_axis(vals_from, indices[:, :, None], axis=1)
    return refs, vals, indices


def kernel(memory, w, b, v, ws, bs, vs):
    lin0 = (jnp.einsum('bsd,da->bsa', memory, w) + b) * v
    probs0 = jax.nn.softmax(jnp.sum(lin0, -1), axis=-1)
    _, vals, ind1 = _topk_select(probs0, memory, _TOP_K1)
    probs = []
    attns = []
    for h in range(_NUM_HEADS):
        lin1 = (jnp.einsum('bkd,da->bka', vals, ws[h]) + bs[h]) * vs[h]
        prob = jax.nn.softmax(jnp.sum(lin1, -1), axis=-1)
        _, attn, _ = _topk_select(prob, lin1, _TOP_K2)
        probs.append(prob)
        attns.append(attn)
    probs = jnp.transpose(jnp.stack(probs), (1, 0, 2))
    attns = jnp.transpose(jnp.stack(attns), (1, 0, 2, 3))
    attns = jnp.max(attns, axis=2)
    return attns, probs, ind1


# speedup vs baseline: 1.0197x; 1.0197x over previous
import jax
import jax.numpy as jnp
from jax.experimental import pallas as pl

_TOP_K1 = 1024
_TOP_K2 = 512


def kernel(memory, w, b, v, ws, bs, vs):
    lin0 = (jnp.einsum('bsd,da->bsa', memory, w) + b) * v
    probs0 = jax.nn.softmax(jnp.sum(lin0, -1), axis=-1)
    _, ind1 = jax.lax.top_k(probs0, _TOP_K1)
    vals = jnp.take_along_axis(memory, ind1[:, :, None], axis=1)
    U = jnp.einsum('hda,ha->dh', ws, vs)
    c = jnp.sum(bs * vs, axis=-1)
    score1 = jnp.einsum('bkd,dh->bhk', vals, U) + c[None, :, None]
    probs = jax.nn.softmax(score1, axis=-1)
    ind2 = jax.lax.top_k(probs, _TOP_K2)[1]
    sel = jnp.take_along_axis(vals[:, None], ind2[..., None], axis=2)
    lin = (jnp.einsum('bhkd,hda->bhka', sel, ws) + bs[None, :, None, :]) * vs[None, :, None, :]
    attns = jnp.max(lin, axis=2)
    return attns, probs, ind1


# speedup vs baseline: 1.3775x; 1.3509x over previous
import functools

import jax
import jax.numpy as jnp
from jax.experimental import pallas as pl

_TOP_K1 = 1024
_TOP_K2 = 512
_INT_MIN = -2147483648


def _l2_body(vals_ref, ws_ref, bs_ref, vs_ref, attns_ref, probs_ref, *, k2):
    x = vals_ref[0]
    wh = ws_ref[0]
    lin = (jnp.dot(x, wh) + bs_ref[0]) * vs_ref[0]
    s = jnp.sum(lin, axis=-1)

    m = jnp.max(s)
    e = jnp.exp(s - m)
    probs_ref[0, 0, 0, :] = e / jnp.sum(e)

    bits = jax.lax.bitcast_convert_type(s, jnp.int32)
    key = jnp.where(bits >= 0, bits, bits ^ jnp.int32(0x7FFFFFFF))
    nneg = jnp.sum((key >= 0).astype(jnp.int32))
    t0 = jnp.where(nneg >= k2, jnp.int32(0), jnp.int32(_INT_MIN))

    def step(i, t):
        cand = t + (jnp.int32(1) << (jnp.int32(30) - i))
        cnt = jnp.sum((key >= cand).astype(jnp.int32))
        return jnp.where(cnt >= k2, cand, t)

    t = jax.lax.fori_loop(0, 31, step, t0)
    mask = key >= t
    sel = jnp.where(mask[:, None], lin, jnp.float32(-jnp.inf))
    attns_ref[0, 0, 0, :] = jnp.max(sel, axis=0)


def _level2(vals, ws, bs, vs, k2):
    B, K1, D = vals.shape
    H, _, A = ws.shape
    bs2 = bs.reshape(H, 1, A)
    vs2 = vs.reshape(H, 1, A)
    attns4, probs4 = pl.pallas_call(
        functools.partial(_l2_body, k2=k2),
        grid=(B, H),
        in_specs=[
            pl.BlockSpec((1, K1, D), lambda b, h: (b, 0, 0)),
            pl.BlockSpec((1, D, A), lambda b, h: (h, 0, 0)),
            pl.BlockSpec((1, 1, A), lambda b, h: (h, 0, 0)),
            pl.BlockSpec((1, 1, A), lambda b, h: (h, 0, 0)),
        ],
        out_specs=[
            pl.BlockSpec((1, 1, 1, A), lambda b, h: (b, h, 0, 0)),
            pl.BlockSpec((1, 1, 1, K1), lambda b, h: (b, h, 0, 0)),
        ],
        out_shape=[
            jax.ShapeDtypeStruct((B, H, 1, A), jnp.float32),
            jax.ShapeDtypeStruct((B, H, 1, K1), jnp.float32),
        ],
    )(vals, ws, bs2, vs2)
    return attns4.reshape(B, H, A), probs4.reshape(B, H, K1)


def kernel(memory, w, b, v, ws, bs, vs):
    B, S, D = memory.shape
    k1 = min(_TOP_K1, S)
    k2 = min(_TOP_K2, k1)
    lin0 = (jnp.einsum('bsd,da->bsa', memory, w) + b) * v
    probs0 = jax.nn.softmax(jnp.sum(lin0, -1), axis=-1)
    _, ind1 = jax.lax.top_k(probs0, k1)
    vals = jnp.take_along_axis(memory, ind1[:, :, None], axis=1)
    attns, probs = _level2(vals, ws, bs, vs, k2)
    return attns, probs, ind1


# speedup vs baseline: 1.3920x; 1.0105x over previous
import functools

import jax
import jax.numpy as jnp
from jax.experimental import pallas as pl

_TOP_K1 = 1024
_TOP_K2 = 512
_INT_MIN = -2147483648


def _l2_body(vals_ref, ws_ref, bs_ref, vs_ref, attns_ref, probs_ref, *, k2):
    wh = ws_ref[0]
    nb = vals_ref.shape[0]
    for bb in range(nb):
        x = vals_ref[bb]
        lin = (jnp.dot(x, wh) + bs_ref[0]) * vs_ref[0]
        s = jnp.sum(lin, axis=-1)

        m = jnp.max(s)
        e = jnp.exp(s - m)
        probs_ref[0, bb, :] = e / jnp.sum(e)

        bits = jax.lax.bitcast_convert_type(s, jnp.int32)
        key = jnp.where(bits >= 0, bits, bits ^ jnp.int32(0x7FFFFFFF))
        nneg = jnp.sum((key >= 0).astype(jnp.int32))
        t0 = jnp.where(nneg >= k2, jnp.int32(0), jnp.int32(_INT_MIN))

        def step(i, t):
            cand = t + (jnp.int32(1) << (jnp.int32(30) - i))
            cnt = jnp.sum((key >= cand).astype(jnp.int32))
            return jnp.where(cnt >= k2, cand, t)

        t = jax.lax.fori_loop(0, 31, step, t0)
        mask = key >= t
        sel = jnp.where(mask[:, None], lin, jnp.float32(-jnp.inf))
        attns_ref[0, bb, :] = jnp.max(sel, axis=0)


def _level2(vals, ws, bs, vs, k2):
    B, K1, D = vals.shape
    H, _, A = ws.shape
    bs2 = bs.reshape(H, 1, A)
    vs2 = vs.reshape(H, 1, A)
    attns4, probs4 = pl.pallas_call(
        functools.partial(_l2_body, k2=k2),
        grid=(H,),
        in_specs=[
            pl.BlockSpec((B, K1, D), lambda h: (0, 0, 0)),
            pl.BlockSpec((1, D, A), lambda h: (h, 0, 0)),
            pl.BlockSpec((1, 1, A), lambda h: (h, 0, 0)),
            pl.BlockSpec((1, 1, A), lambda h: (h, 0, 0)),
        ],
        out_specs=[
            pl.BlockSpec((1, B, A), lambda h: (h, 0, 0)),
            pl.BlockSpec((1, B, K1), lambda h: (h, 0, 0)),
        ],
        out_shape=[
            jax.ShapeDtypeStruct((H, B, A), jnp.float32),
            jax.ShapeDtypeStruct((H, B, K1), jnp.float32),
        ],
    )(vals, ws, bs2, vs2)
    return (jnp.transpose(attns4, (1, 0, 2)),
            jnp.transpose(probs4, (1, 0, 2)))


def kernel(memory, w, b, v, ws, bs, vs):
    B, S, D = memory.shape
    k1 = min(_TOP_K1, S)
    k2 = min(_TOP_K2, k1)
    lin0 = (jnp.einsum('bsd,da->bsa', memory, w) + b) * v
    probs0 = jax.nn.softmax(jnp.sum(lin0, -1), axis=-1)
    _, ind1 = jax.lax.top_k(probs0, k1)
    vals = jnp.take_along_axis(memory, ind1[:, :, None], axis=1)
    attns, probs = _level2(vals, ws, bs, vs, k2)
    return attns, probs, ind1


# speedup vs baseline: 3.2134x; 2.3085x over previous
import functools

import jax
import jax.numpy as jnp
from jax.experimental import pallas as pl

_TOP_K1 = 1024
_TOP_K2 = 512
_INT_MIN = -2147483648


def _l2_body(vals_ref, ws_ref, bs_ref, vs_ref, attns_ref, probs_ref, *, k2):
    wh = ws_ref[0]
    nb = vals_ref.shape[0]
    lins, ss = [], []
    for bb in range(nb):
        x = vals_ref[bb]
        lin = (jnp.dot(x, wh) + bs_ref[0]) * vs_ref[0]
        lins.append(lin)
        ss.append(jnp.sum(lin, axis=-1))

    s = jnp.stack(ss, axis=0)
    m = jnp.max(s, axis=1, keepdims=True)
    e = jnp.exp(s - m)
    probs_ref[0, :, :] = e / jnp.sum(e, axis=1, keepdims=True)

    bits = jax.lax.bitcast_convert_type(s, jnp.int32)
    key = jnp.where(bits >= 0, bits, bits ^ jnp.int32(0x7FFFFFFF))
    nneg = jnp.sum((key >= 0).astype(jnp.int32), axis=1, keepdims=True)
    t = jnp.where(nneg >= k2, jnp.int32(0), jnp.int32(_INT_MIN))
    for i in range(31):
        cand = t + jnp.int32(1 << (30 - i))
        cnt = jnp.sum((key >= cand).astype(jnp.int32), axis=1, keepdims=True)
        t = jnp.where(cnt >= k2, cand, t)

    for bb in range(nb):
        bits_b = jax.lax.bitcast_convert_type(ss[bb], jnp.int32)
        key_b = jnp.where(bits_b >= 0, bits_b, bits_b ^ jnp.int32(0x7FFFFFFF))
        mask = key_b >= t[bb, 0]
        sel = jnp.where(mask[:, None], lins[bb], jnp.float32(-jnp.inf))
        attns_ref[0, bb, :] = jnp.max(sel, axis=0)


def _level2(vals, ws, bs, vs, k2):
    B, K1, D = vals.shape
    H, _, A = ws.shape
    bs2 = bs.reshape(H, 1, A)
    vs2 = vs.reshape(H, 1, A)
    attns4, probs4 = pl.pallas_call(
        functools.partial(_l2_body, k2=k2),
        grid=(H,),
        in_specs=[
            pl.BlockSpec((B, K1, D), lambda h: (0, 0, 0)),
            pl.BlockSpec((1, D, A), lambda h: (h, 0, 0)),
            pl.BlockSpec((1, 1, A), lambda h: (h, 0, 0)),
            pl.BlockSpec((1, 1, A), lambda h: (h, 0, 0)),
        ],
        out_specs=[
            pl.BlockSpec((1, B, A), lambda h: (h, 0, 0)),
            pl.BlockSpec((1, B, K1), lambda h: (h, 0, 0)),
        ],
        out_shape=[
            jax.ShapeDtypeStruct((H, B, A), jnp.float32),
            jax.ShapeDtypeStruct((H, B, K1), jnp.float32),
        ],
    )(vals, ws, bs2, vs2)
    return (jnp.transpose(attns4, (1, 0, 2)),
            jnp.transpose(probs4, (1, 0, 2)))


def kernel(memory, w, b, v, ws, bs, vs):
    B, S, D = memory.shape
    k1 = min(_TOP_K1, S)
    k2 = min(_TOP_K2, k1)
    lin0 = (jnp.einsum('bsd,da->bsa', memory, w) + b) * v
    probs0 = jax.nn.softmax(jnp.sum(lin0, -1), axis=-1)
    _, ind1 = jax.lax.top_k(probs0, k1)
    vals = jnp.take_along_axis(memory, ind1[:, :, None], axis=1)
    attns, probs = _level2(vals, ws, bs, vs, k2)
    return attns, probs, ind1


# speedup vs baseline: 3.3633x; 1.0466x over previous
import functools

import jax
import jax.numpy as jnp
from jax import lax
from jax.experimental import pallas as pl
from jax.experimental.pallas import tpu as pltpu, tpu_sc as plsc

_TOP_K1 = 1024
_TOP_K2 = 512
_INT_MIN = -2147483648


def _l2_body(vals_ref, ws_ref, bs_ref, vs_ref, attns_ref, probs_ref, *, k2):
    wh = ws_ref[0]
    nb = vals_ref.shape[0]
    lins, ss = [], []
    for bb in range(nb):
        x = vals_ref[bb]
        lin = (jnp.dot(x, wh) + bs_ref[0]) * vs_ref[0]
        lins.append(lin)
        ss.append(jnp.sum(lin, axis=-1))

    s = jnp.stack(ss, axis=0)
    m = jnp.max(s, axis=1, keepdims=True)
    e = jnp.exp(s - m)
    probs_ref[0, :, :] = e / jnp.sum(e, axis=1, keepdims=True)

    bits = jax.lax.bitcast_convert_type(s, jnp.int32)
    key = jnp.where(bits >= 0, bits, bits ^ jnp.int32(0x7FFFFFFF))
    nneg = jnp.sum((key >= 0).astype(jnp.int32), axis=1, keepdims=True)
    t = jnp.where(nneg >= k2, jnp.int32(0), jnp.int32(_INT_MIN))
    for i in range(31):
        cand = t + jnp.int32(1 << (30 - i))
        cnt = jnp.sum((key >= cand).astype(jnp.int32), axis=1, keepdims=True)
        t = jnp.where(cnt >= k2, cand, t)

    for bb in range(nb):
        bits_b = jax.lax.bitcast_convert_type(ss[bb], jnp.int32)
        key_b = jnp.where(bits_b >= 0, bits_b, bits_b ^ jnp.int32(0x7FFFFFFF))
        mask = key_b >= t[bb, 0]
        sel = jnp.where(mask[:, None], lins[bb], jnp.float32(-jnp.inf))
        attns_ref[0, bb, :] = jnp.max(sel, axis=0)


def _level2(vals, ws, bs, vs, k2):
    B, K1, D = vals.shape
    H, _, A = ws.shape
    bs2 = bs.reshape(H, 1, A)
    vs2 = vs.reshape(H, 1, A)
    attns4, probs4 = pl.pallas_call(
        functools.partial(_l2_body, k2=k2),
        grid=(H,),
        in_specs=[
            pl.BlockSpec((B, K1, D), lambda h: (0, 0, 0)),
            pl.BlockSpec((1, D, A), lambda h: (h, 0, 0)),
            pl.BlockSpec((1, 1, A), lambda h: (h, 0, 0)),
            pl.BlockSpec((1, 1, A), lambda h: (h, 0, 0)),
        ],
        out_specs=[
            pl.BlockSpec((1, B, A), lambda h: (h, 0, 0)),
            pl.BlockSpec((1, B, K1), lambda h: (h, 0, 0)),
        ],
        out_shape=[
            jax.ShapeDtypeStruct((H, B, A), jnp.float32),
            jax.ShapeDtypeStruct((H, B, K1), jnp.float32),
        ],
    )(vals, ws, bs2, vs2)
    return (jnp.transpose(attns4, (1, 0, 2)),
            jnp.transpose(probs4, (1, 0, 2)))


def _sc_gather_rows(table, idx):
    info = plsc.get_sparse_core_info()
    nw = info.num_cores * info.num_subcores
    n, d = idx.shape[0], table.shape[1]
    b_per_w = n // nw
    ch = min(b_per_w, 64)
    mesh = plsc.VectorSubcoreMesh(core_axis_name="c", subcore_axis_name="s")

    @functools.partial(
        pl.kernel, mesh=mesh,
        out_type=jax.ShapeDtypeStruct((n, d), jnp.float32),
        scratch_types=[
            pltpu.VMEM((ch,), jnp.int32),
            pltpu.VMEM((ch, d), jnp.float32),
            pltpu.SemaphoreType.DMA,
        ],
    )
    def k(table_hbm, idx_hbm, out_hbm, idx_v, rows_v, sem):
        wid = lax.axis_index("s") * info.num_cores + lax.axis_index("c")
        base = wid * b_per_w
        for c in range(b_per_w // ch):
            off = base + c * ch
            pltpu.sync_copy(idx_hbm.at[pl.ds(off, ch)], idx_v)
            pltpu.async_copy(table_hbm.at[idx_v], rows_v, sem).wait()
            pltpu.sync_copy(rows_v, out_hbm.at[pl.ds(off, ch)])

    return k(table, idx)


def kernel(memory, w, b, v, ws, bs, vs):
    B, S, D = memory.shape
    k1 = min(_TOP_K1, S)
    k2 = min(_TOP_K2, k1)
    lin0 = (jnp.einsum('bsd,da->bsa', memory, w) + b) * v
    probs0 = jax.nn.softmax(jnp.sum(lin0, -1), axis=-1)
    _, ind1 = jax.lax.top_k(probs0, k1)
    flat_idx = (ind1 + (jnp.arange(B, dtype=ind1.dtype) * S)[:, None]).reshape(-1)
    vals = _sc_gather_rows(memory.reshape(B * S, D), flat_idx).reshape(B, k1, D)
    attns, probs = _level2(vals, ws, bs, vs, k2)
    return attns, probs, ind1
